# Initial kernel scaffold; baseline (speedup 1.0000x reference)
import jax
import jax.numpy as jnp
from jax.experimental import pallas as pl


def kernel(predicted_boxes, predicted_class_dist, predicted_objectness, boxes, labels, anchor_priors):
    raise NotImplementedError("write your pallas kernel here")


# speedup vs baseline: 8.8512x; 8.8512x over previous
import functools

import jax
import jax.numpy as jnp
from jax import lax
from jax.experimental import pallas as pl
from jax.experimental.pallas import tpu as pltpu

N, P, M, NUM_CLASSES = 16, 16384, 20, 81
NEG_POS_RATIO, ALPHA, THRESHOLD = 3, 1.0, 0.5
PC = 4096
NCHUNK = P // PC
PB = 512
BIG = 1e9


def _fiota(shape, dim):
    return lax.broadcasted_iota(jnp.int32, shape, dim).astype(jnp.float32)


def _match_kernel(boxes_ref, labels_ref, priors_ref, pboxes_ref, pobj_ref,
                  scal_ref, tcn_ref,
                  ioufp_s, ofp_s, neg_s, posf_s, acc_s):
    i = pl.program_id(0)

    @pl.when(i == 0)
    def _init():
        acc_s[0] = 0.0
        acc_s[1] = 0.0

    boxes = boxes_ref[0]
    bx0 = boxes[:, 0:1]
    by0 = boxes[:, 1:2]
    bx1 = boxes[:, 2:3]
    by1 = boxes[:, 3:4]
    area_b = (bx1 - bx0) * (by1 - by0)
    bcx = (bx0 + bx1) * 0.5
    bcy = (by0 + by1) * 0.5
    bw = bx1 - bx0
    bh = by1 - by0
    labels = labels_ref[0]
    jcol = _fiota((M, 1), 0)

    best_v = jnp.full((M, 1), -1.0, jnp.float32)
    best_p = jnp.zeros((M, 1), jnp.float32)

    for c in range(NCHUNK):
        sl = slice(c * PC, (c + 1) * PC)
        px0 = priors_ref[0:1, sl]
        py0 = priors_ref[1:2, sl]
        px1 = priors_ref[2:3, sl]
        py1 = priors_ref[3:4, sl]
        wx = jnp.maximum(jnp.minimum(bx1, px1) - jnp.maximum(bx0, px0), 0.0)
        wy = jnp.maximum(jnp.minimum(by1, py1) - jnp.maximum(by0, py0), 0.0)
        inter = wx * wy
        area_p = (px1 - px0) * (py1 - py0)
        iou = inter / (area_b + area_p - inter)
        fp = jnp.max(iou, axis=0, keepdims=True)
        jmat = _fiota((M, PC), 0)
        ofp_c = jnp.min(jnp.where(iou == fp, jmat, BIG), axis=0, keepdims=True)
        ioufp_s[0:1, sl] = fp
        ofp_s[0:1, sl] = ofp_c
        cm = jnp.max(iou, axis=1, keepdims=True)
        pmat = _fiota((M, PC), 1) + float(c * PC)
        cam = jnp.min(jnp.where(iou == cm, pmat, BIG), axis=1, keepdims=True)
        upd = cm > best_v
        best_v = jnp.where(upd, cm, best_v)
        best_p = jnp.where(upd, cam, best_p)

    pfo = best_p

    loc_sum = jnp.float32(0.0)
    obj_pos = jnp.float32(0.0)
    pb_all = pboxes_ref[0]

    for c in range(NCHUNK):
        sl = slice(c * PC, (c + 1) * PC)
        fp = ioufp_s[0:1, sl]
        ofp_c = ofp_s[0:1, sl]
        pg = _fiota((1, PC), 1) + float(c * PC)
        match = pfo == pg
        fj = jnp.max(jnp.where(match, jcol, -1.0), axis=0, keepdims=True)
        has = fj >= 0.0
        ofp2 = jnp.where(has, fj, ofp_c)
        iou2 = jnp.where(has, 1.0, fp)
        posf = (iou2 >= THRESHOLD).astype(jnp.float32)
        onehot = (ofp2 == jcol).astype(jnp.float32)
        lab = jnp.sum(onehot * labels, axis=0, keepdims=True)
        tcn_ref[0, 0:1, sl] = jnp.where(posf > 0.0, lab, -1.0)

        ocx = jnp.sum(onehot * bcx, axis=0, keepdims=True)
        ocy = jnp.sum(onehot * bcy, axis=0, keepdims=True)
        ow = jnp.sum(onehot * bw, axis=0, keepdims=True)
        oh = jnp.sum(onehot * bh, axis=0, keepdims=True)
        px0 = priors_ref[0:1, sl]
        py0 = priors_ref[1:2, sl]
        px1 = priors_ref[2:3, sl]
        py1 = priors_ref[3:4, sl]
        pcx = (px0 + px1) * 0.5
        pcy = (py0 + py1) * 0.5
        pw = px1 - px0
        ph = py1 - py0
        g = jnp.concatenate([
            (ocx - pcx) / (pw * 0.1),
            (ocy - pcy) / (ph * 0.1),
            jnp.log(ow / pw) * 5.0,
            jnp.log(oh / ph) * 5.0,
        ], axis=0)
        d = pb_all[:, sl] - g
        ad = jnp.abs(d)
        sl1 = jnp.where(ad < 1.0, 0.5 * d * d, ad - 0.5)
        loc_sum = loc_sum + jnp.sum(sl1 * posf)

        po = pobj_ref[0, 0:1, sl]
        oel = (po - posf) ** 2
        obj_pos = obj_pos + jnp.sum(oel * posf)
        neg_s[pl.ds(i, 1), sl] = jnp.where(posf > 0.0, 0.0, oel)
        posf_s[pl.ds(i, 1), sl] = posf

    acc_s[0] = acc_s[0] + loc_sum
    acc_s[1] = acc_s[1] + obj_pos

    @pl.when(i == N - 1)
    def _finalize():
        negs = neg_s[:, :]
        posv = posf_s[:, :]
        npv = jnp.sum(posv, axis=1, keepdims=True)
        kv = jnp.minimum(jnp.float32(NEG_POS_RATIO) * npv, jnp.float32(P))
        bits = lax.bitcast_convert_type(negs, jnp.int32)
        lo = jnp.zeros((N, 1), jnp.int32)
        hi = jnp.max(bits, axis=1, keepdims=True)
        for _ in range(31):
            mid = lo + lax.div(hi - lo + 1, 2)
            cnt = jnp.sum((bits >= mid).astype(jnp.float32), axis=1,
                          keepdims=True)
            ok = cnt >= kv
            lo = jnp.where(ok, mid, lo)
            hi = jnp.where(ok, hi, mid - 1)
        tval = jnp.max(jnp.where(bits <= lo, negs, -1.0), axis=1, keepdims=True)
        cgt = jnp.sum(jnp.where(bits > lo, 1.0, 0.0), axis=1, keepdims=True)
        sgt = jnp.sum(jnp.where(bits > lo, negs, 0.0), axis=1, keepdims=True)
        topk = sgt + (kv - cgt) * tval
        topk = jnp.where(kv > 0.0, topk, 0.0)
        lane = _fiota((1, 128), 1)
        out = (jnp.where(lane == 0.0, acc_s[0], 0.0)
               + jnp.where(lane == 1.0, acc_s[1], 0.0)
               + jnp.where(lane == 2.0, jnp.sum(topk), 0.0)
               + jnp.where(lane == 3.0, jnp.sum(npv), 0.0)
               + jnp.where(lane == 4.0, jnp.sum(kv), 0.0))
        scal_ref[:, :] = out


def _class_kernel(cd_ref, tc_ref, out_ref):
    @pl.when((pl.program_id(0) == 0) & (pl.program_id(1) == 0))
    def _init():
        out_ref[:, :] = jnp.zeros_like(out_ref)

    x = cd_ref[0]
    m = jnp.max(x, axis=1, keepdims=True)
    lse = m + jnp.log(jnp.sum(jnp.exp(x - m), axis=1, keepdims=True))
    tc = tc_ref[0]
    cid = _fiota((PB, NUM_CLASSES), 1)
    take = jnp.sum(jnp.where(cid == tc, x, 0.0), axis=1, keepdims=True)
    nll = lse - take
    partial = jnp.sum(jnp.where(tc >= 0.0, nll, 0.0))
    lane = _fiota((1, 128), 1)
    out_ref[:, :] += jnp.where(lane == 0.0, partial, 0.0)


@jax.jit
def kernel(predicted_boxes, predicted_class_dist, predicted_objectness,
           boxes, labels, anchor_priors):
    priors_t = anchor_priors.T
    pboxes_t = jnp.transpose(predicted_boxes, (0, 2, 1))
    pobj = predicted_objectness.reshape(N, 1, P)
    labels_f = labels.astype(jnp.float32).reshape(N, M, 1)

    scal, tcn = pl.pallas_call(
        _match_kernel,
        grid=(N,),
        in_specs=[
            pl.BlockSpec((1, M, 4), lambda i: (i, 0, 0)),
            pl.BlockSpec((1, M, 1), lambda i: (i, 0, 0)),
            pl.BlockSpec((4, P), lambda i: (0, 0)),
            pl.BlockSpec((1, 4, P), lambda i: (i, 0, 0)),
            pl.BlockSpec((1, 1, P), lambda i: (i, 0, 0)),
        ],
        out_specs=[
            pl.BlockSpec((1, 128), lambda i: (0, 0)),
            pl.BlockSpec((1, 1, P), lambda i: (i, 0, 0)),
        ],
        out_shape=[
            jax.ShapeDtypeStruct((1, 128), jnp.float32),
            jax.ShapeDtypeStruct((N, 1, P), jnp.float32),
        ],
        scratch_shapes=[
            pltpu.VMEM((1, P), jnp.float32),
            pltpu.VMEM((1, P), jnp.float32),
            pltpu.VMEM((N, P), jnp.float32),
            pltpu.VMEM((N, P), jnp.float32),
            pltpu.SMEM((4,), jnp.float32),
        ],
    )(boxes, labels_f, priors_t, pboxes_t, pobj)

    tcn_col = tcn.reshape(N, P, 1)
    scal2 = pl.pallas_call(
        _class_kernel,
        grid=(N, P // PB),
        in_specs=[
            pl.BlockSpec((1, PB, NUM_CLASSES), lambda i, j: (i, j, 0)),
            pl.BlockSpec((1, PB, 1), lambda i, j: (i, j, 0)),
        ],
        out_specs=pl.BlockSpec((1, 128), lambda i, j: (0, 0)),
        out_shape=jax.ShapeDtypeStruct((1, 128), jnp.float32),
    )(predicted_class_dist, tcn_col)

    loc_sum = scal[0, 0]
    obj_pos = scal[0, 1]
    neg_topk = scal[0, 2]
    n_pos = scal[0, 3]
    k_tot = scal[0, 4]
    class_sum = scal2[0, 0]
    return (class_sum / n_pos + obj_pos / n_pos
            + neg_topk / jnp.maximum(k_tot, 1.0)
            + ALPHA * loc_sum / (n_pos * 4.0))
